# baseline (speedup 1.0000x reference)
import functools

import jax
import jax.numpy as jnp
from jax import lax
from jax.experimental import pallas as pl
from jax.experimental.pallas import tpu as pltpu
from jax.experimental.pallas import tpu_sc as plsc

N = 10000
NP = 10240
F = 128
E = 320000
NC, NS, L = 2, 16, 16
NW = NC * NS
C = 128
CHUNKS = 80
EPW = C * CHUNKS
EP = EPW * NW
RPT = NP // NS
NB = 2
K = 1
G = 16
NG = CHUNKS // G
HC = 128
HCH = 80

_mesh = plsc.VectorSubcoreMesh(core_axis_name="c", subcore_axis_name="s")


@functools.partial(
    pl.kernel,
    out_type=jax.ShapeDtypeStruct((NC, 2, NP), jnp.float32),
    mesh=_mesh,
    scratch_types=[
        pltpu.VMEM((HCH, HC), jnp.int32),
        pltpu.VMEM((HCH, HC), jnp.int32),
        pltpu.VMEM((HC,), jnp.float32),
        pltpu.VMEM((RPT,), jnp.float32),
        pltpu.VMEM_SHARED((NP,), jnp.float32),
        pltpu.VMEM_SHARED((NP,), jnp.float32),
        pltpu.SemaphoreType.DMA,
        pltpu.SemaphoreType.DMA,
    ],
)
def _hist_kernel(src_hbm, dst_hbm, hist_hbm, sidx_v, didx_v, ones_v,
                 zero_v, hsrc_sh, hdst_sh, ssem, dsem):
    cid = lax.axis_index("c")
    sid = lax.axis_index("s")
    w = cid * NS + sid

    def fill_ones(i, _):
        ones_v[pl.ds(i * L, L)] = jnp.full((L,), 1.0, jnp.float32)
        return 0

    lax.fori_loop(0, HC // L, fill_ones, 0)

    def fill_zero(i, _):
        zero_v[pl.ds(i * L, L)] = jnp.zeros((L,), jnp.float32)
        return 0

    lax.fori_loop(0, RPT // L, fill_zero, 0)

    pltpu.sync_copy(src_hbm.at[w], sidx_v)
    pltpu.sync_copy(dst_hbm.at[w], didx_v)

    pltpu.sync_copy(zero_v, hsrc_sh.at[pl.ds(sid * RPT, RPT)])
    pltpu.sync_copy(zero_v, hdst_sh.at[pl.ds(sid * RPT, RPT)])
    plsc.subcore_barrier()

    BATCH = 8

    def body(i, _):
        for b in range(BATCH):
            ch = i * BATCH + b
            pltpu.async_copy(ones_v, hsrc_sh.at[sidx_v.at[ch]], ssem,
                             add=True)
            pltpu.async_copy(ones_v, hdst_sh.at[didx_v.at[ch]], dsem,
                             add=True)
        for b in range(BATCH):
            pltpu.make_async_copy(ones_v, hsrc_sh.at[pl.ds(0, HC)],
                                  ssem).wait()
            pltpu.make_async_copy(ones_v, hdst_sh.at[pl.ds(0, HC)],
                                  dsem).wait()
        return 0

    lax.fori_loop(0, HCH // BATCH, body, 0)
    plsc.subcore_barrier()

    pltpu.sync_copy(hsrc_sh.at[pl.ds(sid * RPT, RPT)],
                    hist_hbm.at[cid, 0, pl.ds(sid * RPT, RPT)])
    pltpu.sync_copy(hdst_sh.at[pl.ds(sid * RPT, RPT)],
                    hist_hbm.at[cid, 1, pl.ds(sid * RPT, RPT)])


@functools.partial(
    pl.kernel,
    out_type=jax.ShapeDtypeStruct((NC, NP, F), jnp.float32),
    mesh=_mesh,
    scratch_types=[
        pltpu.VMEM((2, G, C), jnp.int32),
        pltpu.VMEM((2, G, C), jnp.int32),
        pltpu.VMEM((NB, C, F), jnp.float32),
        pltpu.SemaphoreType.DMA((NB,)),
        pltpu.SemaphoreType.DMA((NB,)),
        pltpu.SemaphoreType.DMA((2,)),
        pltpu.SemaphoreType.DMA,
        pltpu.VMEM_SHARED((NP, F), jnp.float32),
    ],
)
def _scatter_kernel(src_hbm, dst_hbm, xw_hbm, out_hbm, sidx_v, didx_v,
                    rows_v, gsem, ssem, isem, zsem, acc_sh):
    cid = lax.axis_index("c")
    sid = lax.axis_index("s")
    w = cid * NS + sid

    for k in range(RPT // C):
        pltpu.async_copy(xw_hbm.at[pl.ds(N, C)],
                         acc_sh.at[pl.ds(sid * RPT + k * C, C)], zsem)

    def load_idx(g, s, sync):
        if sync:
            pltpu.sync_copy(src_hbm.at[w, pl.ds(g * G, G)], sidx_v.at[s])
            pltpu.sync_copy(dst_hbm.at[w, pl.ds(g * G, G)], didx_v.at[s])
        else:
            pltpu.async_copy(src_hbm.at[w, pl.ds(g * G, G)],
                             sidx_v.at[s], isem.at[s])
            pltpu.async_copy(dst_hbm.at[w, pl.ds(g * G, G)],
                             didx_v.at[s], isem.at[s])

    def wait_idx(s):
        pltpu.make_async_copy(src_hbm.at[w, pl.ds(0, G)], sidx_v.at[s],
                              isem.at[s]).wait()
        pltpu.make_async_copy(dst_hbm.at[w, pl.ds(0, G)], didx_v.at[s],
                              isem.at[s]).wait()

    load_idx(0, 0, True)
    load_idx(1, 1, False)
    for k in range(RPT // C):
        pltpu.make_async_copy(xw_hbm.at[pl.ds(N, C)],
                              acc_sh.at[pl.ds(0, C)], zsem).wait()
    plsc.subcore_barrier()

    def gather(ch, b):
        g, r = divmod(ch, G)
        pltpu.async_copy(xw_hbm.at[sidx_v.at[g % 2, r]], rows_v.at[b],
                         gsem.at[b])

    def gather_wait(b):
        pltpu.make_async_copy(xw_hbm.at[pl.ds(0, C)], rows_v.at[b],
                              gsem.at[b]).wait()

    def scatter(ch, b):
        g, r = divmod(ch, G)
        pltpu.async_copy(rows_v.at[b], acc_sh.at[didx_v.at[g % 2, r]],
                         ssem.at[b], add=True)

    def scatter_wait(b):
        pltpu.make_async_copy(rows_v.at[b], acc_sh.at[pl.ds(0, C)],
                              ssem.at[b]).wait()

    for b in range(NB):
        gather(b, b)
    waited = []
    for ch in range(CHUNKS):
        gather_wait(ch % NB)
        scatter(ch, ch % NB)
        gg = ch // G
        if ch % G == K and 1 <= gg and gg + 1 < NG:
            load_idx(gg + 1, (gg + 1) % 2, False)
        old = ch - K
        nxt = old + NB
        if old >= 0 and nxt < CHUNKS:
            scatter_wait(old % NB)
            waited.append(old)
            if nxt % G == 0:
                wait_idx((nxt // G) % 2)
            gather(nxt, old % NB)
    for old in range(CHUNKS):
        if old not in waited:
            scatter_wait(old % NB)
    plsc.subcore_barrier()
    pltpu.sync_copy(acc_sh.at[pl.ds(sid * RPT, RPT)],
                    out_hbm.at[cid, pl.ds(sid * RPT, RPT)])


BR = 512


def _l1_body(hist_ref, h_ref, w_ref, o_ref):
    deg = hist_ref[0, 0] + hist_ref[1, 0]
    ns = lax.rsqrt(jnp.maximum(deg, 1.0))[:, None]
    o_ref[...] = jnp.dot(h_ref[...] * ns, w_ref[...],
                         preferred_element_type=jnp.float32)


def _l2_body(hist_ref, agg_ref, b_ref, w_ref, o_ref):
    di = hist_ref[0, 1] + hist_ref[1, 1]
    do = hist_ref[0, 0] + hist_ref[1, 0]
    nd = lax.rsqrt(jnp.maximum(di, 1.0))[:, None]
    ns = lax.rsqrt(jnp.maximum(do, 1.0))[:, None]
    a = agg_ref[0] + agg_ref[1]
    x = jnp.maximum(a * nd + b_ref[...], 0.0)
    o_ref[...] = jnp.dot(x * ns, w_ref[...],
                         preferred_element_type=jnp.float32)


def _l3_body(hist_ref, agg_ref, b_ref, o_ref):
    di = hist_ref[0, 1] + hist_ref[1, 1]
    nd = lax.rsqrt(jnp.maximum(di, 1.0))[:, None]
    o_ref[...] = (agg_ref[0] + agg_ref[1]) * nd + b_ref[...]


_l1_call = pl.pallas_call(
    _l1_body,
    grid=(NP // BR,),
    in_specs=[
        pl.BlockSpec((NC, 2, BR), lambda i: (0, 0, i)),
        pl.BlockSpec((BR, F), lambda i: (i, 0)),
        pl.BlockSpec((F, F), lambda i: (0, 0)),
    ],
    out_specs=pl.BlockSpec((BR, F), lambda i: (i, 0)),
    out_shape=jax.ShapeDtypeStruct((NP, F), jnp.float32),
)

_l2_call = pl.pallas_call(
    _l2_body,
    grid=(NP // BR,),
    in_specs=[
        pl.BlockSpec((NC, 2, BR), lambda i: (0, 0, i)),
        pl.BlockSpec((NC, BR, F), lambda i: (0, i, 0)),
        pl.BlockSpec((1, F), lambda i: (0, 0)),
        pl.BlockSpec((F, F), lambda i: (0, 0)),
    ],
    out_specs=pl.BlockSpec((BR, F), lambda i: (i, 0)),
    out_shape=jax.ShapeDtypeStruct((NP, F), jnp.float32),
)

_l3_call = pl.pallas_call(
    _l3_body,
    grid=(NP // BR,),
    in_specs=[
        pl.BlockSpec((NC, 2, BR), lambda i: (0, 0, i)),
        pl.BlockSpec((NC, BR, F), lambda i: (0, i, 0)),
        pl.BlockSpec((1, F), lambda i: (0, 0)),
    ],
    out_specs=pl.BlockSpec((BR, F), lambda i: (i, 0)),
    out_shape=jax.ShapeDtypeStruct((NP, F), jnp.float32),
)


@jax.jit
def kernel(h, edge_index, W1, b1, W2, b2):
    e = edge_index.astype(jnp.int32)
    src = e[0]
    dst = e[1]
    pad = N + (jnp.arange(EP - E, dtype=jnp.int32) % (NP - N))
    src_p = jnp.concatenate([src, pad]).reshape(NW, CHUNKS, C)
    dst_p = jnp.concatenate([dst, pad]).reshape(NW, CHUNKS, C)
    src_h = src_p.reshape(NW, HCH, HC)
    dst_h = dst_p.reshape(NW, HCH, HC)
    h_pad = jnp.zeros((NP, F), jnp.float32).at[:N].set(h)

    hist = _hist_kernel(src_h, dst_h)
    xw1 = _l1_call(hist, h_pad, W1)
    agg1 = _scatter_kernel(src_p, dst_p, xw1)
    xw2 = _l2_call(hist, agg1, b1.reshape(1, F), W2)
    agg2 = _scatter_kernel(src_p, dst_p, xw2)
    out = _l3_call(hist, agg2, b2.reshape(1, F))
    return out[:N]


# speedup vs baseline: 1.1360x; 1.1360x over previous
import functools

import jax
import jax.numpy as jnp
from jax import lax
from jax.experimental import pallas as pl
from jax.experimental.pallas import tpu as pltpu
from jax.experimental.pallas import tpu_sc as plsc

N = 10000
NP = 10240
F = 128
E = 320000
NC, NS, L = 2, 16, 16
NW = NC * NS
C = 128
CHUNKS = 80
EPW = C * CHUNKS
EP = EPW * NW
RPT = NP // NS
NB = 2
K = 1
G = 16
NG = CHUNKS // G
HC = 128
HCH = 80

_mesh = plsc.VectorSubcoreMesh(core_axis_name="c", subcore_axis_name="s")


@functools.partial(
    pl.kernel,
    out_type=jax.ShapeDtypeStruct((NC, 2, NP), jnp.float32),
    mesh=_mesh,
    scratch_types=[
        pltpu.VMEM((HCH, HC), jnp.int32),
        pltpu.VMEM((HCH, HC), jnp.int32),
        pltpu.VMEM((HC,), jnp.float32),
        pltpu.VMEM((RPT,), jnp.float32),
        pltpu.VMEM_SHARED((NP,), jnp.float32),
        pltpu.VMEM_SHARED((NP,), jnp.float32),
        pltpu.SemaphoreType.DMA,
        pltpu.SemaphoreType.DMA,
    ],
)
def _hist_kernel(src_hbm, dst_hbm, hist_hbm, sidx_v, didx_v, ones_v,
                 zero_v, hsrc_sh, hdst_sh, ssem, dsem):
    cid = lax.axis_index("c")
    sid = lax.axis_index("s")
    w = cid * NS + sid

    def fill_ones(i, _):
        ones_v[pl.ds(i * L, L)] = jnp.full((L,), 1.0, jnp.float32)
        return 0

    lax.fori_loop(0, HC // L, fill_ones, 0)

    def fill_zero(i, _):
        zero_v[pl.ds(i * L, L)] = jnp.zeros((L,), jnp.float32)
        return 0

    lax.fori_loop(0, RPT // L, fill_zero, 0)

    pltpu.sync_copy(src_hbm.at[w], sidx_v)
    pltpu.sync_copy(dst_hbm.at[w], didx_v)

    pltpu.sync_copy(zero_v, hsrc_sh.at[pl.ds(sid * RPT, RPT)])
    pltpu.sync_copy(zero_v, hdst_sh.at[pl.ds(sid * RPT, RPT)])
    plsc.subcore_barrier()

    BATCH = 8

    def body(i, _):
        for b in range(BATCH):
            ch = i * BATCH + b
            pltpu.async_copy(ones_v, hsrc_sh.at[sidx_v.at[ch]], ssem,
                             add=True)
            pltpu.async_copy(ones_v, hdst_sh.at[didx_v.at[ch]], dsem,
                             add=True)
        for b in range(BATCH):
            pltpu.make_async_copy(ones_v, hsrc_sh.at[pl.ds(0, HC)],
                                  ssem).wait()
            pltpu.make_async_copy(ones_v, hdst_sh.at[pl.ds(0, HC)],
                                  dsem).wait()
        return 0

    lax.fori_loop(0, HCH // BATCH, body, 0)
    plsc.subcore_barrier()

    pltpu.sync_copy(hsrc_sh.at[pl.ds(sid * RPT, RPT)],
                    hist_hbm.at[cid, 0, pl.ds(sid * RPT, RPT)])
    pltpu.sync_copy(hdst_sh.at[pl.ds(sid * RPT, RPT)],
                    hist_hbm.at[cid, 1, pl.ds(sid * RPT, RPT)])


@functools.partial(
    pl.kernel,
    out_type=jax.ShapeDtypeStruct((NC, NP, F), jnp.float32),
    mesh=_mesh,
    scratch_types=[
        pltpu.VMEM((2, G, C), jnp.int32),
        pltpu.VMEM((2, G, C), jnp.int32),
        pltpu.VMEM((NB, C, F), jnp.float32),
        pltpu.SemaphoreType.DMA((NB,)),
        pltpu.SemaphoreType.DMA((2,)),
        pltpu.SemaphoreType.DMA,
        pltpu.VMEM_SHARED((NP, F), jnp.float32),
    ],
)
def _scatter_kernel(src_hbm, dst_hbm, xw_hbm, out_hbm, sidx_v, didx_v,
                    rows_v, gsem, isem, zsem, acc_sh):
    cid = lax.axis_index("c")
    sid = lax.axis_index("s")
    w = cid * NS + sid

    for k in range(RPT // C):
        pltpu.async_copy(xw_hbm.at[pl.ds(N, C)],
                         acc_sh.at[pl.ds(sid * RPT + k * C, C)], zsem)

    def load_idx(g, s, sync):
        if sync:
            pltpu.sync_copy(src_hbm.at[w, pl.ds(g * G, G)], sidx_v.at[s])
            pltpu.sync_copy(dst_hbm.at[w, pl.ds(g * G, G)], didx_v.at[s])
        else:
            pltpu.async_copy(src_hbm.at[w, pl.ds(g * G, G)],
                             sidx_v.at[s], isem.at[s])
            pltpu.async_copy(dst_hbm.at[w, pl.ds(g * G, G)],
                             didx_v.at[s], isem.at[s])

    def wait_idx(s):
        pltpu.make_async_copy(src_hbm.at[w, pl.ds(0, G)], sidx_v.at[s],
                              isem.at[s]).wait()
        pltpu.make_async_copy(dst_hbm.at[w, pl.ds(0, G)], didx_v.at[s],
                              isem.at[s]).wait()

    load_idx(0, 0, True)
    load_idx(1, 1, False)
    for k in range(RPT // C):
        pltpu.make_async_copy(xw_hbm.at[pl.ds(N, C)],
                              acc_sh.at[pl.ds(0, C)], zsem).wait()
    plsc.subcore_barrier()

    def gather(ch, b):
        g, r = divmod(ch, G)
        pltpu.async_copy(xw_hbm.at[sidx_v.at[g % 2, r]], rows_v.at[b],
                         gsem.at[b])

    def gather_wait(b):
        pltpu.make_async_copy(xw_hbm.at[pl.ds(0, C)], rows_v.at[b],
                              gsem.at[b]).wait()

    def scatter(ch, b):
        g, r = divmod(ch, G)
        pltpu.sync_copy(rows_v.at[b], acc_sh.at[didx_v.at[g % 2, r]],
                        add=True)

    gather(0, 0)
    gather(1, 1)
    for g in range(NG):
        for r in range(G):
            ch = g * G + r
            b = ch % NB
            gather_wait(b)
            scatter(ch, b)
            nxt = ch + NB
            if nxt < CHUNKS:
                if nxt % G == 0:
                    wait_idx((nxt // G) % 2)
                gather(nxt, b)
        if g + 2 < NG:
            load_idx(g + 2, g % 2, False)
    plsc.subcore_barrier()
    pltpu.sync_copy(acc_sh.at[pl.ds(sid * RPT, RPT)],
                    out_hbm.at[cid, pl.ds(sid * RPT, RPT)])


BR = 512


def _l1_body(hist_ref, h_ref, w_ref, o_ref):
    deg = hist_ref[0, 0] + hist_ref[1, 0]
    ns = lax.rsqrt(jnp.maximum(deg, 1.0))[:, None]
    o_ref[...] = jnp.dot(h_ref[...] * ns, w_ref[...],
                         preferred_element_type=jnp.float32)


def _l2_body(hist_ref, agg_ref, b_ref, w_ref, o_ref):
    di = hist_ref[0, 1] + hist_ref[1, 1]
    do = hist_ref[0, 0] + hist_ref[1, 0]
    nd = lax.rsqrt(jnp.maximum(di, 1.0))[:, None]
    ns = lax.rsqrt(jnp.maximum(do, 1.0))[:, None]
    a = agg_ref[0] + agg_ref[1]
    x = jnp.maximum(a * nd + b_ref[...], 0.0)
    o_ref[...] = jnp.dot(x * ns, w_ref[...],
                         preferred_element_type=jnp.float32)


def _l3_body(hist_ref, agg_ref, b_ref, o_ref):
    di = hist_ref[0, 1] + hist_ref[1, 1]
    nd = lax.rsqrt(jnp.maximum(di, 1.0))[:, None]
    o_ref[...] = (agg_ref[0] + agg_ref[1]) * nd + b_ref[...]


_l1_call = pl.pallas_call(
    _l1_body,
    grid=(NP // BR,),
    in_specs=[
        pl.BlockSpec((NC, 2, BR), lambda i: (0, 0, i)),
        pl.BlockSpec((BR, F), lambda i: (i, 0)),
        pl.BlockSpec((F, F), lambda i: (0, 0)),
    ],
    out_specs=pl.BlockSpec((BR, F), lambda i: (i, 0)),
    out_shape=jax.ShapeDtypeStruct((NP, F), jnp.float32),
)

_l2_call = pl.pallas_call(
    _l2_body,
    grid=(NP // BR,),
    in_specs=[
        pl.BlockSpec((NC, 2, BR), lambda i: (0, 0, i)),
        pl.BlockSpec((NC, BR, F), lambda i: (0, i, 0)),
        pl.BlockSpec((1, F), lambda i: (0, 0)),
        pl.BlockSpec((F, F), lambda i: (0, 0)),
    ],
    out_specs=pl.BlockSpec((BR, F), lambda i: (i, 0)),
    out_shape=jax.ShapeDtypeStruct((NP, F), jnp.float32),
)

_l3_call = pl.pallas_call(
    _l3_body,
    grid=(NP // BR,),
    in_specs=[
        pl.BlockSpec((NC, 2, BR), lambda i: (0, 0, i)),
        pl.BlockSpec((NC, BR, F), lambda i: (0, i, 0)),
        pl.BlockSpec((1, F), lambda i: (0, 0)),
    ],
    out_specs=pl.BlockSpec((BR, F), lambda i: (i, 0)),
    out_shape=jax.ShapeDtypeStruct((NP, F), jnp.float32),
)


@jax.jit
def kernel(h, edge_index, W1, b1, W2, b2):
    e = edge_index.astype(jnp.int32)
    src = e[0]
    dst = e[1]
    pad = N + (jnp.arange(EP - E, dtype=jnp.int32) % (NP - N))
    src_p = jnp.concatenate([src, pad]).reshape(NW, CHUNKS, C)
    dst_p = jnp.concatenate([dst, pad]).reshape(NW, CHUNKS, C)
    src_h = src_p.reshape(NW, HCH, HC)
    dst_h = dst_p.reshape(NW, HCH, HC)
    h_pad = jnp.zeros((NP, F), jnp.float32).at[:N].set(h)

    hist = _hist_kernel(src_h, dst_h)
    xw1 = _l1_call(hist, h_pad, W1)
    agg1 = _scatter_kernel(src_p, dst_p, xw1)
    xw2 = _l2_call(hist, agg1, b1.reshape(1, F), W2)
    agg2 = _scatter_kernel(src_p, dst_p, xw2)
    out = _l3_call(hist, agg2, b2.reshape(1, F))
    return out[:N]


# speedup vs baseline: 1.2224x; 1.0761x over previous
import functools

import jax
import jax.numpy as jnp
from jax import lax
from jax.experimental import pallas as pl
from jax.experimental.pallas import tpu as pltpu
from jax.experimental.pallas import tpu_sc as plsc

N = 10000
NP = 10240
F = 128
E = 320000
NC, NS, L = 2, 16, 16
NW = NC * NS
C = 128
CHUNKS = 80
EPW = C * CHUNKS
EP = EPW * NW
RPT = NP // NS
NB = 2
K = 1
G = 16
NG = CHUNKS // G
HC = 128
HCH = 80

_mesh = plsc.VectorSubcoreMesh(core_axis_name="c", subcore_axis_name="s")


@functools.partial(
    pl.kernel,
    out_type=jax.ShapeDtypeStruct((NC, 2, NP), jnp.float32),
    mesh=_mesh,
    scratch_types=[
        pltpu.VMEM((HCH, HC), jnp.int32),
        pltpu.VMEM((HCH, HC), jnp.int32),
        pltpu.VMEM((HC,), jnp.float32),
        pltpu.VMEM((RPT,), jnp.float32),
        pltpu.VMEM_SHARED((NP,), jnp.float32),
        pltpu.VMEM_SHARED((NP,), jnp.float32),
        pltpu.SemaphoreType.DMA,
        pltpu.SemaphoreType.DMA,
    ],
)
def _hist_kernel(src_hbm, dst_hbm, hist_hbm, sidx_v, didx_v, ones_v,
                 zero_v, hsrc_sh, hdst_sh, ssem, dsem):
    cid = lax.axis_index("c")
    sid = lax.axis_index("s")
    w = cid * NS + sid

    def fill_ones(i, _):
        ones_v[pl.ds(i * L, L)] = jnp.full((L,), 1.0, jnp.float32)
        return 0

    lax.fori_loop(0, HC // L, fill_ones, 0)

    def fill_zero(i, _):
        zero_v[pl.ds(i * L, L)] = jnp.zeros((L,), jnp.float32)
        return 0

    lax.fori_loop(0, RPT // L, fill_zero, 0)

    pltpu.sync_copy(src_hbm.at[w], sidx_v)
    pltpu.sync_copy(dst_hbm.at[w], didx_v)

    pltpu.sync_copy(zero_v, hsrc_sh.at[pl.ds(sid * RPT, RPT)])
    pltpu.sync_copy(zero_v, hdst_sh.at[pl.ds(sid * RPT, RPT)])
    plsc.subcore_barrier()

    BATCH = 8

    def body(i, _):
        for b in range(BATCH):
            ch = i * BATCH + b
            pltpu.async_copy(ones_v, hsrc_sh.at[sidx_v.at[ch]], ssem,
                             add=True)
            pltpu.async_copy(ones_v, hdst_sh.at[didx_v.at[ch]], dsem,
                             add=True)
        for b in range(BATCH):
            pltpu.make_async_copy(ones_v, hsrc_sh.at[pl.ds(0, HC)],
                                  ssem).wait()
            pltpu.make_async_copy(ones_v, hdst_sh.at[pl.ds(0, HC)],
                                  dsem).wait()
        return 0

    lax.fori_loop(0, HCH // BATCH, body, 0)
    plsc.subcore_barrier()

    pltpu.sync_copy(hsrc_sh.at[pl.ds(sid * RPT, RPT)],
                    hist_hbm.at[cid, 0, pl.ds(sid * RPT, RPT)])
    pltpu.sync_copy(hdst_sh.at[pl.ds(sid * RPT, RPT)],
                    hist_hbm.at[cid, 1, pl.ds(sid * RPT, RPT)])


@functools.partial(
    pl.kernel,
    out_type=jax.ShapeDtypeStruct((NC, NP, F), jnp.float32),
    mesh=_mesh,
    scratch_types=[
        pltpu.VMEM((2, G, C), jnp.int32),
        pltpu.VMEM((2, G, C), jnp.int32),
        pltpu.VMEM((NB, C, F), jnp.float32),
        pltpu.SemaphoreType.DMA((NB,)),
        pltpu.SemaphoreType.DMA((2,)),
        pltpu.SemaphoreType.DMA,
        pltpu.VMEM_SHARED((NP, F), jnp.float32),
    ],
)
def _scatter_kernel(src_hbm, dst_hbm, xw_hbm, out_hbm, sidx_v, didx_v,
                    rows_v, gsem, isem, zsem, acc_sh):
    cid = lax.axis_index("c")
    sid = lax.axis_index("s")
    w = cid * NS + sid

    def fill_zero(r, _):
        for cc in range(F // L):
            rows_v[0, r, pl.ds(cc * L, L)] = jnp.zeros((L,), jnp.float32)
        return 0

    lax.fori_loop(0, C, fill_zero, 0)
    for k in range(RPT // C):
        pltpu.async_copy(rows_v.at[0],
                         acc_sh.at[pl.ds(sid * RPT + k * C, C)], zsem)

    def load_idx(g, s, sync):
        if sync:
            pltpu.sync_copy(src_hbm.at[w, pl.ds(g * G, G)], sidx_v.at[s])
            pltpu.sync_copy(dst_hbm.at[w, pl.ds(g * G, G)], didx_v.at[s])
        else:
            pltpu.async_copy(src_hbm.at[w, pl.ds(g * G, G)],
                             sidx_v.at[s], isem.at[s])
            pltpu.async_copy(dst_hbm.at[w, pl.ds(g * G, G)],
                             didx_v.at[s], isem.at[s])

    def wait_idx(s):
        pltpu.make_async_copy(src_hbm.at[w, pl.ds(0, G)], sidx_v.at[s],
                              isem.at[s]).wait()
        pltpu.make_async_copy(dst_hbm.at[w, pl.ds(0, G)], didx_v.at[s],
                              isem.at[s]).wait()

    load_idx(0, 0, True)
    load_idx(1, 1, False)
    for k in range(RPT // C):
        pltpu.make_async_copy(rows_v.at[0], acc_sh.at[pl.ds(0, C)],
                              zsem).wait()
    plsc.subcore_barrier()

    def gather(ch, b):
        g, r = divmod(ch, G)
        pltpu.async_copy(xw_hbm.at[sidx_v.at[g % 2, r]], rows_v.at[b],
                         gsem.at[b])

    def gather_wait(b):
        pltpu.make_async_copy(xw_hbm.at[pl.ds(0, C)], rows_v.at[b],
                              gsem.at[b]).wait()

    def scatter(ch, b):
        g, r = divmod(ch, G)
        pltpu.sync_copy(rows_v.at[b], acc_sh.at[didx_v.at[g % 2, r]],
                        add=True)

    gather(0, 0)
    gather(1, 1)
    for g in range(NG):
        for r in range(G):
            ch = g * G + r
            b = ch % NB
            gather_wait(b)
            scatter(ch, b)
            nxt = ch + NB
            if nxt < CHUNKS:
                if nxt % G == 0:
                    wait_idx((nxt // G) % 2)
                gather(nxt, b)
        if g + 2 < NG:
            load_idx(g + 2, g % 2, False)
    plsc.subcore_barrier()
    pltpu.sync_copy(acc_sh.at[pl.ds(sid * RPT, RPT)],
                    out_hbm.at[cid, pl.ds(sid * RPT, RPT)])


BR = 512


def _l1_body(hist_ref, h_ref, w_ref, o_ref):
    deg = hist_ref[0, 0] + hist_ref[1, 0]
    ns = lax.rsqrt(jnp.maximum(deg, 1.0))[:, None]
    o_ref[...] = jnp.dot(h_ref[...] * ns, w_ref[...],
                         preferred_element_type=jnp.float32)


def _l2_body(hist_ref, agg_ref, b_ref, w_ref, o_ref):
    di = hist_ref[0, 1] + hist_ref[1, 1]
    do = hist_ref[0, 0] + hist_ref[1, 0]
    nd = lax.rsqrt(jnp.maximum(di, 1.0))[:, None]
    ns = lax.rsqrt(jnp.maximum(do, 1.0))[:, None]
    a = agg_ref[0] + agg_ref[1]
    x = jnp.maximum(a * nd + b_ref[...], 0.0)
    o_ref[...] = jnp.dot(x * ns, w_ref[...],
                         preferred_element_type=jnp.float32)


def _l3_body(hist_ref, agg_ref, b_ref, o_ref):
    di = hist_ref[0, 1] + hist_ref[1, 1]
    nd = lax.rsqrt(jnp.maximum(di, 1.0))[:, None]
    o_ref[...] = (agg_ref[0] + agg_ref[1]) * nd + b_ref[...]


_l1_call = pl.pallas_call(
    _l1_body,
    grid=(NP // BR,),
    in_specs=[
        pl.BlockSpec((NC, 2, BR), lambda i: (0, 0, i)),
        pl.BlockSpec((BR, F), lambda i: (i, 0)),
        pl.BlockSpec((F, F), lambda i: (0, 0)),
    ],
    out_specs=pl.BlockSpec((BR, F), lambda i: (i, 0)),
    out_shape=jax.ShapeDtypeStruct((NP, F), jnp.float32),
)

_l2_call = pl.pallas_call(
    _l2_body,
    grid=(NP // BR,),
    in_specs=[
        pl.BlockSpec((NC, 2, BR), lambda i: (0, 0, i)),
        pl.BlockSpec((NC, BR, F), lambda i: (0, i, 0)),
        pl.BlockSpec((1, F), lambda i: (0, 0)),
        pl.BlockSpec((F, F), lambda i: (0, 0)),
    ],
    out_specs=pl.BlockSpec((BR, F), lambda i: (i, 0)),
    out_shape=jax.ShapeDtypeStruct((NP, F), jnp.float32),
)

_l3_call = pl.pallas_call(
    _l3_body,
    grid=(NP // BR,),
    in_specs=[
        pl.BlockSpec((NC, 2, BR), lambda i: (0, 0, i)),
        pl.BlockSpec((NC, BR, F), lambda i: (0, i, 0)),
        pl.BlockSpec((1, F), lambda i: (0, 0)),
    ],
    out_specs=pl.BlockSpec((BR, F), lambda i: (i, 0)),
    out_shape=jax.ShapeDtypeStruct((NP, F), jnp.float32),
)


@jax.jit
def kernel(h, edge_index, W1, b1, W2, b2):
    e = edge_index.astype(jnp.int32)
    src = e[0]
    dst = e[1]
    pad = N + (jnp.arange(EP - E, dtype=jnp.int32) % (NP - N))
    src_p = jnp.concatenate([src, pad]).reshape(NW, CHUNKS, C)
    dst_p = jnp.concatenate([dst, pad]).reshape(NW, CHUNKS, C)
    src_h = src_p.reshape(NW, HCH, HC)
    dst_h = dst_p.reshape(NW, HCH, HC)
    h_pad = jnp.zeros((NP, F), jnp.float32).at[:N].set(h)

    hist = _hist_kernel(src_h, dst_h)
    xw1 = _l1_call(hist, h_pad, W1)
    agg1 = _scatter_kernel(src_p, dst_p, xw1)
    xw2 = _l2_call(hist, agg1, b1.reshape(1, F), W2)
    agg2 = _scatter_kernel(src_p, dst_p, xw2)
    out = _l3_call(hist, agg2, b2.reshape(1, F))
    return out[:N]


# speedup vs baseline: 1.2855x; 1.0517x over previous
import functools

import jax
import jax.numpy as jnp
from jax import lax
from jax.experimental import pallas as pl
from jax.experimental.pallas import tpu as pltpu
from jax.experimental.pallas import tpu_sc as plsc

N = 10000
NP = 10240
F = 128
E = 320000
NC, NS, L = 2, 16, 16
NW = NC * NS
C = 128
CHUNKS = 80
EPW = C * CHUNKS
EP = EPW * NW
RPT = NP // NS
NB = 2
K = 1
G = 16
NG = CHUNKS // G
HC = 128
HCH = 80

_mesh = plsc.VectorSubcoreMesh(core_axis_name="c", subcore_axis_name="s")


@functools.partial(
    pl.kernel,
    out_type=jax.ShapeDtypeStruct((NC, 2, NP), jnp.float32),
    mesh=_mesh,
    scratch_types=[
        pltpu.VMEM((HCH, HC), jnp.int32),
        pltpu.VMEM((HCH, HC), jnp.int32),
        pltpu.VMEM((HC,), jnp.float32),
        pltpu.VMEM((RPT,), jnp.float32),
        pltpu.VMEM_SHARED((NP,), jnp.float32),
        pltpu.VMEM_SHARED((NP,), jnp.float32),
        pltpu.SemaphoreType.DMA,
        pltpu.SemaphoreType.DMA,
    ],
)
def _hist_kernel(src_hbm, dst_hbm, hist_hbm, sidx_v, didx_v, ones_v,
                 zero_v, hsrc_sh, hdst_sh, ssem, dsem):
    cid = lax.axis_index("c")
    sid = lax.axis_index("s")
    w = cid * NS + sid

    def fill_ones(i, _):
        ones_v[pl.ds(i * L, L)] = jnp.full((L,), 1.0, jnp.float32)
        return 0

    lax.fori_loop(0, HC // L, fill_ones, 0)

    def fill_zero(i, _):
        zero_v[pl.ds(i * L, L)] = jnp.zeros((L,), jnp.float32)
        return 0

    lax.fori_loop(0, RPT // L, fill_zero, 0)

    pltpu.sync_copy(src_hbm.at[w], sidx_v)
    pltpu.sync_copy(dst_hbm.at[w], didx_v)

    pltpu.sync_copy(zero_v, hsrc_sh.at[pl.ds(sid * RPT, RPT)])
    pltpu.sync_copy(zero_v, hdst_sh.at[pl.ds(sid * RPT, RPT)])
    plsc.subcore_barrier()

    BATCH = 16

    def body(i, _):
        for b in range(BATCH):
            ch = i * BATCH + b
            pltpu.async_copy(ones_v, hsrc_sh.at[sidx_v.at[ch]], ssem,
                             add=True)
            pltpu.async_copy(ones_v, hdst_sh.at[didx_v.at[ch]], dsem,
                             add=True)
        for b in range(BATCH):
            pltpu.make_async_copy(ones_v, hsrc_sh.at[pl.ds(0, HC)],
                                  ssem).wait()
            pltpu.make_async_copy(ones_v, hdst_sh.at[pl.ds(0, HC)],
                                  dsem).wait()
        return 0

    lax.fori_loop(0, HCH // BATCH, body, 0)
    plsc.subcore_barrier()

    pltpu.sync_copy(hsrc_sh.at[pl.ds(sid * RPT, RPT)],
                    hist_hbm.at[cid, 0, pl.ds(sid * RPT, RPT)])
    pltpu.sync_copy(hdst_sh.at[pl.ds(sid * RPT, RPT)],
                    hist_hbm.at[cid, 1, pl.ds(sid * RPT, RPT)])


@functools.partial(
    pl.kernel,
    out_type=jax.ShapeDtypeStruct((NC, NP, F), jnp.float32),
    mesh=_mesh,
    scratch_types=[
        pltpu.VMEM((2, G, C), jnp.int32),
        pltpu.VMEM((2, G, C), jnp.int32),
        pltpu.VMEM((NB, C, F), jnp.float32),
        pltpu.SemaphoreType.DMA((NB,)),
        pltpu.SemaphoreType.DMA((2,)),
        pltpu.SemaphoreType.DMA,
        pltpu.VMEM_SHARED((NP, F), jnp.float32),
    ],
)
def _scatter_kernel(src_hbm, dst_hbm, xw_hbm, out_hbm, sidx_v, didx_v,
                    rows_v, gsem, isem, zsem, acc_sh):
    cid = lax.axis_index("c")
    sid = lax.axis_index("s")
    w = cid * NS + sid

    def fill_zero(r, _):
        for cc in range(F // L):
            rows_v[0, r, pl.ds(cc * L, L)] = jnp.zeros((L,), jnp.float32)
        return 0

    lax.fori_loop(0, C, fill_zero, 0)
    for k in range(RPT // C):
        pltpu.async_copy(rows_v.at[0],
                         acc_sh.at[pl.ds(sid * RPT + k * C, C)], zsem)

    def load_idx(g, s, sync):
        if sync:
            pltpu.sync_copy(src_hbm.at[w, pl.ds(g * G, G)], sidx_v.at[s])
            pltpu.sync_copy(dst_hbm.at[w, pl.ds(g * G, G)], didx_v.at[s])
        else:
            pltpu.async_copy(src_hbm.at[w, pl.ds(g * G, G)],
                             sidx_v.at[s], isem.at[s])
            pltpu.async_copy(dst_hbm.at[w, pl.ds(g * G, G)],
                             didx_v.at[s], isem.at[s])

    def wait_idx(s):
        pltpu.make_async_copy(src_hbm.at[w, pl.ds(0, G)], sidx_v.at[s],
                              isem.at[s]).wait()
        pltpu.make_async_copy(dst_hbm.at[w, pl.ds(0, G)], didx_v.at[s],
                              isem.at[s]).wait()

    load_idx(0, 0, True)
    load_idx(1, 1, False)
    for k in range(RPT // C):
        pltpu.make_async_copy(rows_v.at[0], acc_sh.at[pl.ds(0, C)],
                              zsem).wait()
    plsc.subcore_barrier()

    def gather(ch, b):
        g, r = divmod(ch, G)
        pltpu.async_copy(xw_hbm.at[sidx_v.at[g % 2, r]], rows_v.at[b],
                         gsem.at[b])

    def gather_wait(b):
        pltpu.make_async_copy(xw_hbm.at[pl.ds(0, C)], rows_v.at[b],
                              gsem.at[b]).wait()

    def scatter(ch, b):
        g, r = divmod(ch, G)
        pltpu.sync_copy(rows_v.at[b], acc_sh.at[didx_v.at[g % 2, r]],
                        add=True)

    gather(0, 0)
    gather(1, 1)
    for g in range(NG):
        for r in range(G):
            ch = g * G + r
            b = ch % NB
            gather_wait(b)
            scatter(ch, b)
            nxt = ch + NB
            if nxt < CHUNKS:
                if nxt % G == 0:
                    wait_idx((nxt // G) % 2)
                gather(nxt, b)
        if g + 2 < NG:
            load_idx(g + 2, g % 2, False)
    plsc.subcore_barrier()
    pltpu.sync_copy(acc_sh.at[pl.ds(sid * RPT, RPT)],
                    out_hbm.at[cid, pl.ds(sid * RPT, RPT)])


BR = 1024


def _l1_body(hist_ref, h_ref, w_ref, o_ref):
    deg = hist_ref[0, 0] + hist_ref[1, 0]
    ns = lax.rsqrt(jnp.maximum(deg, 1.0))[:, None]
    o_ref[...] = jnp.dot(h_ref[...] * ns, w_ref[...],
                         preferred_element_type=jnp.float32)


def _l2_body(hist_ref, agg_ref, b_ref, w_ref, o_ref):
    di = hist_ref[0, 1] + hist_ref[1, 1]
    do = hist_ref[0, 0] + hist_ref[1, 0]
    nd = lax.rsqrt(jnp.maximum(di, 1.0))[:, None]
    ns = lax.rsqrt(jnp.maximum(do, 1.0))[:, None]
    a = agg_ref[0] + agg_ref[1]
    x = jnp.maximum(a * nd + b_ref[...], 0.0)
    o_ref[...] = jnp.dot(x * ns, w_ref[...],
                         preferred_element_type=jnp.float32)


def _l3_body(hist_ref, agg_ref, b_ref, o_ref):
    di = hist_ref[0, 1] + hist_ref[1, 1]
    nd = lax.rsqrt(jnp.maximum(di, 1.0))[:, None]
    o_ref[...] = (agg_ref[0] + agg_ref[1]) * nd + b_ref[...]


_l1_call = pl.pallas_call(
    _l1_body,
    grid=(NP // BR,),
    in_specs=[
        pl.BlockSpec((NC, 2, BR), lambda i: (0, 0, i)),
        pl.BlockSpec((BR, F), lambda i: (i, 0)),
        pl.BlockSpec((F, F), lambda i: (0, 0)),
    ],
    out_specs=pl.BlockSpec((BR, F), lambda i: (i, 0)),
    out_shape=jax.ShapeDtypeStruct((NP, F), jnp.float32),
)

_l2_call = pl.pallas_call(
    _l2_body,
    grid=(NP // BR,),
    in_specs=[
        pl.BlockSpec((NC, 2, BR), lambda i: (0, 0, i)),
        pl.BlockSpec((NC, BR, F), lambda i: (0, i, 0)),
        pl.BlockSpec((1, F), lambda i: (0, 0)),
        pl.BlockSpec((F, F), lambda i: (0, 0)),
    ],
    out_specs=pl.BlockSpec((BR, F), lambda i: (i, 0)),
    out_shape=jax.ShapeDtypeStruct((NP, F), jnp.float32),
)

_l3_call = pl.pallas_call(
    _l3_body,
    grid=(NP // BR,),
    in_specs=[
        pl.BlockSpec((NC, 2, BR), lambda i: (0, 0, i)),
        pl.BlockSpec((NC, BR, F), lambda i: (0, i, 0)),
        pl.BlockSpec((1, F), lambda i: (0, 0)),
    ],
    out_specs=pl.BlockSpec((BR, F), lambda i: (i, 0)),
    out_shape=jax.ShapeDtypeStruct((NP, F), jnp.float32),
)


@jax.jit
def kernel(h, edge_index, W1, b1, W2, b2):
    e = edge_index.astype(jnp.int32)
    src = e[0]
    dst = e[1]
    pad = N + (jnp.arange(EP - E, dtype=jnp.int32) % (NP - N))
    src_p = jnp.concatenate([src, pad]).reshape(NW, CHUNKS, C)
    dst_p = jnp.concatenate([dst, pad]).reshape(NW, CHUNKS, C)
    src_h = src_p.reshape(NW, HCH, HC)
    dst_h = dst_p.reshape(NW, HCH, HC)
    h_pad = jnp.zeros((NP, F), jnp.float32).at[:N].set(h)

    hist = _hist_kernel(src_h, dst_h)
    xw1 = _l1_call(hist, h_pad, W1)
    agg1 = _scatter_kernel(src_p, dst_p, xw1)
    xw2 = _l2_call(hist, agg1, b1.reshape(1, F), W2)
    agg2 = _scatter_kernel(src_p, dst_p, xw2)
    out = _l3_call(hist, agg2, b2.reshape(1, F))
    return out[:N]


# speedup vs baseline: 1.3145x; 1.0225x over previous
import functools

import jax
import jax.numpy as jnp
from jax import lax
from jax.experimental import pallas as pl
from jax.experimental.pallas import tpu as pltpu
from jax.experimental.pallas import tpu_sc as plsc

N = 10000
NP = 10240
F = 128
E = 320000
NC, NS, L = 2, 16, 16
NW = NC * NS
C = 128
CHUNKS = 80
EPW = C * CHUNKS
EP = EPW * NW
RPT = NP // NS
NB = 2
K = 1
G = 16
NG = CHUNKS // G
HC = 128
HCH = 80

_mesh = plsc.VectorSubcoreMesh(core_axis_name="c", subcore_axis_name="s")


@functools.partial(
    pl.kernel,
    out_type=jax.ShapeDtypeStruct((NC, 2, NP), jnp.float32),
    mesh=_mesh,
    scratch_types=[
        pltpu.VMEM((HCH, HC), jnp.int32),
        pltpu.VMEM((HCH, HC), jnp.int32),
        pltpu.VMEM((HC,), jnp.float32),
        pltpu.VMEM((RPT,), jnp.float32),
        pltpu.VMEM_SHARED((NP,), jnp.float32),
        pltpu.VMEM_SHARED((NP,), jnp.float32),
        pltpu.SemaphoreType.DMA,
        pltpu.SemaphoreType.DMA,
    ],
)
def _hist_kernel(src_hbm, dst_hbm, hist_hbm, sidx_v, didx_v, ones_v,
                 zero_v, hsrc_sh, hdst_sh, ssem, dsem):
    cid = lax.axis_index("c")
    sid = lax.axis_index("s")
    w = cid * NS + sid

    def fill_ones(i, _):
        ones_v[pl.ds(i * L, L)] = jnp.full((L,), 1.0, jnp.float32)
        return 0

    lax.fori_loop(0, HC // L, fill_ones, 0)

    def fill_zero(i, _):
        zero_v[pl.ds(i * L, L)] = jnp.zeros((L,), jnp.float32)
        return 0

    lax.fori_loop(0, RPT // L, fill_zero, 0)

    pltpu.sync_copy(src_hbm.at[w], sidx_v)
    pltpu.sync_copy(dst_hbm.at[w], didx_v)

    pltpu.sync_copy(zero_v, hsrc_sh.at[pl.ds(sid * RPT, RPT)])
    pltpu.sync_copy(zero_v, hdst_sh.at[pl.ds(sid * RPT, RPT)])
    plsc.subcore_barrier()

    BATCH = 40

    def body(i, _):
        for b in range(BATCH):
            ch = i * BATCH + b
            pltpu.async_copy(ones_v, hsrc_sh.at[sidx_v.at[ch]], ssem,
                             add=True)
            pltpu.async_copy(ones_v, hdst_sh.at[didx_v.at[ch]], dsem,
                             add=True)
        for b in range(BATCH):
            pltpu.make_async_copy(ones_v, hsrc_sh.at[pl.ds(0, HC)],
                                  ssem).wait()
            pltpu.make_async_copy(ones_v, hdst_sh.at[pl.ds(0, HC)],
                                  dsem).wait()
        return 0

    lax.fori_loop(0, HCH // BATCH, body, 0)
    plsc.subcore_barrier()

    pltpu.sync_copy(hsrc_sh.at[pl.ds(sid * RPT, RPT)],
                    hist_hbm.at[cid, 0, pl.ds(sid * RPT, RPT)])
    pltpu.sync_copy(hdst_sh.at[pl.ds(sid * RPT, RPT)],
                    hist_hbm.at[cid, 1, pl.ds(sid * RPT, RPT)])


@functools.partial(
    pl.kernel,
    out_type=jax.ShapeDtypeStruct((NC, NP, F), jnp.float32),
    mesh=_mesh,
    scratch_types=[
        pltpu.VMEM((2, G, C), jnp.int32),
        pltpu.VMEM((2, G, C), jnp.int32),
        pltpu.VMEM((NB, C, F), jnp.float32),
        pltpu.SemaphoreType.DMA((NB,)),
        pltpu.SemaphoreType.DMA((2,)),
        pltpu.SemaphoreType.DMA,
        pltpu.VMEM_SHARED((NP, F), jnp.float32),
    ],
)
def _scatter_kernel(src_hbm, dst_hbm, xw_hbm, out_hbm, sidx_v, didx_v,
                    rows_v, gsem, isem, zsem, acc_sh):
    cid = lax.axis_index("c")
    sid = lax.axis_index("s")
    w = cid * NS + sid

    def fill_zero(r, _):
        for cc in range(F // L):
            rows_v[0, r, pl.ds(cc * L, L)] = jnp.zeros((L,), jnp.float32)
        return 0

    lax.fori_loop(0, C, fill_zero, 0)
    for k in range(RPT // C):
        pltpu.async_copy(rows_v.at[0],
                         acc_sh.at[pl.ds(sid * RPT + k * C, C)], zsem)

    def load_idx(g, s, sync):
        if sync:
            pltpu.sync_copy(src_hbm.at[w, pl.ds(g * G, G)], sidx_v.at[s])
            pltpu.sync_copy(dst_hbm.at[w, pl.ds(g * G, G)], didx_v.at[s])
        else:
            pltpu.async_copy(src_hbm.at[w, pl.ds(g * G, G)],
                             sidx_v.at[s], isem.at[s])
            pltpu.async_copy(dst_hbm.at[w, pl.ds(g * G, G)],
                             didx_v.at[s], isem.at[s])

    def wait_idx(s):
        pltpu.make_async_copy(src_hbm.at[w, pl.ds(0, G)], sidx_v.at[s],
                              isem.at[s]).wait()
        pltpu.make_async_copy(dst_hbm.at[w, pl.ds(0, G)], didx_v.at[s],
                              isem.at[s]).wait()

    load_idx(0, 0, True)
    load_idx(1, 1, False)
    for k in range(RPT // C):
        pltpu.make_async_copy(rows_v.at[0], acc_sh.at[pl.ds(0, C)],
                              zsem).wait()
    plsc.subcore_barrier()

    def gather(ch, b):
        g, r = divmod(ch, G)
        pltpu.async_copy(xw_hbm.at[sidx_v.at[g % 2, r]], rows_v.at[b],
                         gsem.at[b])

    def gather_wait(b):
        pltpu.make_async_copy(xw_hbm.at[pl.ds(0, C)], rows_v.at[b],
                              gsem.at[b]).wait()

    def scatter(ch, b):
        g, r = divmod(ch, G)
        pltpu.sync_copy(rows_v.at[b], acc_sh.at[didx_v.at[g % 2, r]],
                        add=True)

    gather(0, 0)
    gather(1, 1)
    for g in range(NG):
        for r in range(G):
            ch = g * G + r
            b = ch % NB
            gather_wait(b)
            scatter(ch, b)
            nxt = ch + NB
            if nxt < CHUNKS:
                if nxt % G == 0:
                    wait_idx((nxt // G) % 2)
                gather(nxt, b)
        if g + 2 < NG:
            load_idx(g + 2, g % 2, False)
    plsc.subcore_barrier()
    pltpu.sync_copy(acc_sh.at[pl.ds(sid * RPT, RPT)],
                    out_hbm.at[cid, pl.ds(sid * RPT, RPT)])


BR = 2048


def _l1_body(hist_ref, h_ref, w_ref, o_ref):
    deg = hist_ref[0, 0] + hist_ref[1, 0]
    ns = lax.rsqrt(jnp.maximum(deg, 1.0))[:, None]
    o_ref[...] = jnp.dot(h_ref[...] * ns, w_ref[...],
                         preferred_element_type=jnp.float32)


def _l2_body(hist_ref, agg_ref, b_ref, w_ref, o_ref):
    di = hist_ref[0, 1] + hist_ref[1, 1]
    do = hist_ref[0, 0] + hist_ref[1, 0]
    nd = lax.rsqrt(jnp.maximum(di, 1.0))[:, None]
    ns = lax.rsqrt(jnp.maximum(do, 1.0))[:, None]
    a = agg_ref[0] + agg_ref[1]
    x = jnp.maximum(a * nd + b_ref[...], 0.0)
    o_ref[...] = jnp.dot(x * ns, w_ref[...],
                         preferred_element_type=jnp.float32)


def _l3_body(hist_ref, agg_ref, b_ref, o_ref):
    di = hist_ref[0, 1] + hist_ref[1, 1]
    nd = lax.rsqrt(jnp.maximum(di, 1.0))[:, None]
    o_ref[...] = (agg_ref[0] + agg_ref[1]) * nd + b_ref[...]


_l1_call = pl.pallas_call(
    _l1_body,
    grid=(NP // BR,),
    in_specs=[
        pl.BlockSpec((NC, 2, BR), lambda i: (0, 0, i)),
        pl.BlockSpec((BR, F), lambda i: (i, 0)),
        pl.BlockSpec((F, F), lambda i: (0, 0)),
    ],
    out_specs=pl.BlockSpec((BR, F), lambda i: (i, 0)),
    out_shape=jax.ShapeDtypeStruct((NP, F), jnp.float32),
)

_l2_call = pl.pallas_call(
    _l2_body,
    grid=(NP // BR,),
    in_specs=[
        pl.BlockSpec((NC, 2, BR), lambda i: (0, 0, i)),
        pl.BlockSpec((NC, BR, F), lambda i: (0, i, 0)),
        pl.BlockSpec((1, F), lambda i: (0, 0)),
        pl.BlockSpec((F, F), lambda i: (0, 0)),
    ],
    out_specs=pl.BlockSpec((BR, F), lambda i: (i, 0)),
    out_shape=jax.ShapeDtypeStruct((NP, F), jnp.float32),
)

_l3_call = pl.pallas_call(
    _l3_body,
    grid=(NP // BR,),
    in_specs=[
        pl.BlockSpec((NC, 2, BR), lambda i: (0, 0, i)),
        pl.BlockSpec((NC, BR, F), lambda i: (0, i, 0)),
        pl.BlockSpec((1, F), lambda i: (0, 0)),
    ],
    out_specs=pl.BlockSpec((BR, F), lambda i: (i, 0)),
    out_shape=jax.ShapeDtypeStruct((NP, F), jnp.float32),
)


@jax.jit
def kernel(h, edge_index, W1, b1, W2, b2):
    e = edge_index.astype(jnp.int32)
    src = e[0]
    dst = e[1]
    pad = N + (jnp.arange(EP - E, dtype=jnp.int32) % (NP - N))
    src_p = jnp.concatenate([src, pad]).reshape(NW, CHUNKS, C)
    dst_p = jnp.concatenate([dst, pad]).reshape(NW, CHUNKS, C)
    src_h = src_p.reshape(NW, HCH, HC)
    dst_h = dst_p.reshape(NW, HCH, HC)
    h_pad = jnp.zeros((NP, F), jnp.float32).at[:N].set(h)

    hist = _hist_kernel(src_h, dst_h)
    xw1 = _l1_call(hist, h_pad, W1)
    agg1 = _scatter_kernel(src_p, dst_p, xw1)
    xw2 = _l2_call(hist, agg1, b1.reshape(1, F), W2)
    agg2 = _scatter_kernel(src_p, dst_p, xw2)
    out = _l3_call(hist, agg2, b2.reshape(1, F))
    return out[:N]
